# baseline (speedup 1.0000x reference)
import jax, jax.numpy as jnp
from jax import lax
from jax.experimental import pallas as pl

N_GRAPHS = 256
N_FILTERS = 32
CONV_OUT_LEN = 121


def _bn(h, g, b):
    mu = jnp.mean(h, axis=0)
    var = jnp.var(h, axis=0)
    return (h - mu) / jnp.sqrt(var + 1e-5) * g + b


def _identity_kernel(x_ref, o_ref):
    o_ref[...] = x_ref[...]


def kernel(x, params, edge_index, batch, target):
    src, dst = edge_index[0], edge_index[1]
    h = x
    for l in range(5):
        agg = jnp.zeros_like(h).at[dst].add(h[src])
        m = h + agg
        m = jnp.maximum(m @ params[f"gin{l}_1_w"] + params[f"gin{l}_1_b"], 0.0)
        m = m @ params[f"gin{l}_2_w"] + params[f"gin{l}_2_b"]
        h = jnp.maximum(m, 0.0)
        h = _bn(h, params[f"bn{l}_g"], params[f"bn{l}_b"])
    pooled = jax.ops.segment_sum(h, batch, num_segments=N_GRAPHS)
    xd = jnp.maximum(pooled @ params["fc1_xd_w"] + params["fc1_xd_b"], 0.0)
    emb = params["emb_xt"][target]
    conv = lax.conv_general_dilated(
        emb, params["conv_xt_w"], window_strides=(1,), padding="VALID",
        dimension_numbers=("NCH", "OIH", "NCH"))
    conv = conv + params["conv_xt_b"][None, :, None]
    xt = conv.reshape(-1, N_FILTERS * CONV_OUT_LEN)
    xt = xt @ params["fc1_xt_w"] + params["fc1_xt_b"]
    xc = jnp.concatenate([xd, xt], axis=1)
    xc = jnp.maximum(xc @ params["fc1_w"] + params["fc1_b"], 0.0)
    xc = jnp.maximum(xc @ params["fc2_w"] + params["fc2_b"], 0.0)
    out = xc @ params["out_w"] + params["out_b"]
    out = pl.pallas_call(
        _identity_kernel,
        out_shape=jax.ShapeDtypeStruct(out.shape, out.dtype),
    )(out)
    return out


# speedup vs baseline: 2.1576x; 2.1576x over previous
import functools

import jax
import jax.numpy as jnp
from jax import lax
from jax.experimental import pallas as pl
from jax.experimental.pallas import tpu as pltpu
from jax.experimental.pallas import tpu_sc as plsc

N_NODES = 50000
N_EDGES = 800000
N_GRAPHS = 256
D_IN = 78
DIM = 32
LANES = 128
EMBED_DIM = 128
OUTPUT_DIM = 128
N_FILTERS = 32
SEQ_LEN = 1000
VOCAB_T = 26
KW = 8
CONV_OUT_LEN = 121

NC = 2
NS = 16
CHUNK = 128
G = 392
SEGROWS = 48
SEGS = tuple((o, min(SEGROWS, G - o)) for o in range(0, G, SEGROWS))
EPAD = NS * G * CHUNK
NPAD = 50176
NSLICE = 4
S = NPAD // NSLICE
ZROWS = S // NS
MBLK = 3136
NBLK = NPAD // MBLK

_HI = lax.Precision.HIGHEST


def _sc_agg_body(kk, comb_hbm, p_hbm, out_hbm, acc):
    pl.run_scoped(
        functools.partial(_sc_agg_inner, kk, comb_hbm, p_hbm,
                          out_hbm, acc),
        pltpu.VMEM((SEGROWS, CHUNK), jnp.int32),
        pltpu.VMEM((SEGROWS, CHUNK), jnp.int32),
        pltpu.VMEM((CHUNK, LANES), jnp.float32),
        pltpu.VMEM((CHUNK,), jnp.int32),
    )


def _sc_agg_inner(kk, comb_hbm, p_hbm, out_hbm, acc,
                  src_v, dst_v, buf, ri0):
    c = lax.axis_index("c")
    s = lax.axis_index("s")
    wid = c * NS + s
    base = wid * 4 * G + kk
    iota16 = lax.iota(jnp.int32, 16)
    zero16 = jnp.zeros((16,), jnp.float32)

    def zbody(r, carry):
        for v in range(LANES // 16):
            buf[r, pl.ds(v * 16, 16)] = zero16
        return carry

    lax.fori_loop(0, CHUNK, zbody, 0)
    nfull = ZROWS // CHUNK
    for t in range(nfull):
        pltpu.sync_copy(buf, acc.at[pl.ds(s * ZROWS + t * CHUNK, CHUNK)])
    rem = ZROWS - nfull * CHUNK
    if rem:
        pltpu.sync_copy(buf.at[pl.ds(0, rem)],
                        acc.at[pl.ds(s * ZROWS + nfull * CHUNK, rem)])
    plsc.subcore_barrier()

    def body(j, carry):
        pltpu.sync_copy(
            p_hbm.at[plsc.Indices(src_v.at[j], ignored_value=-1)], buf)
        pltpu.sync_copy(
            buf, acc.at[plsc.Indices(dst_v.at[j], ignored_value=-1)],
            add=True)
        return carry

    def seg_body(js, carry):
        off = js * SEGROWS
        for a, tgt in ((0, src_v), (1, dst_v)):
            for v in range(SEGROWS // 16):
                rloc = off + v * 16 + iota16
                ri0[pl.ds(v * 16, 16)] = jnp.where(
                    rloc < G, base + 2 * a + 4 * rloc, -1)
            pltpu.sync_copy(
                comb_hbm.at[plsc.Indices(ri0.at[pl.ds(0, SEGROWS)],
                                         ignored_value=-1)], tgt)
        n = jnp.minimum(G - off, SEGROWS)
        lax.fori_loop(0, n, body, 0)
        return carry

    lax.fori_loop(0, -(-G // SEGROWS), seg_body, 0)
    plsc.subcore_barrier()
    for t in range(nfull + (1 if rem else 0)):
        n = CHUNK if t < nfull else rem
        pltpu.sync_copy(acc.at[pl.ds(s * ZROWS + t * CHUNK, n)],
                        buf.at[pl.ds(0, n)])
        orow = c * S + s * ZROWS + t * CHUNK
        for v in range(CHUNK // 16):
            rloc = v * 16 + iota16
            ri0[pl.ds(v * 16, 16)] = jnp.where(rloc < n, orow + rloc, -1)
        pltpu.sync_copy(buf, out_hbm.at[plsc.Indices(ri0, ignored_value=-1)])


@functools.cache
def _make_sc_agg(kk):
    return pl.kernel(
        functools.partial(_sc_agg_body, kk),
        out_type=jax.ShapeDtypeStruct((NC * S, LANES), jnp.float32),
        mesh=plsc.VectorSubcoreMesh(core_axis_name="c", subcore_axis_name="s",
                                    num_cores=NC, num_subcores=NS),
        scratch_types=[
            pltpu.VMEM_SHARED((S, LANES), jnp.float32),
        ],
    )


def _sc_agg(kk, comb, p128):
    return _make_sc_agg(kk)(comb, p128)


def _mid_body(kin, h_ref, a_ref, w1_ref, b1_ref, w2_ref, b2_ref,
              r_ref, st_ref):
    pid = pl.program_id(0)
    u = h_ref[:, :kin] + a_ref[:, :kin]
    m1 = jnp.maximum(jnp.dot(u, w1_ref[...],
                             preferred_element_type=jnp.float32)
                     + b1_ref[...], 0.0)
    v = jnp.dot(m1, w2_ref[...],
                preferred_element_type=jnp.float32) + b2_ref[...]
    r = jnp.maximum(v, 0.0)
    rows = lax.broadcasted_iota(jnp.int32, r.shape, 0) + pid * MBLK
    r = jnp.where(rows < N_NODES, r, 0.0)
    r_ref[...] = r
    s = jnp.sum(r, axis=0)[None, :]
    q = jnp.sum(r * r, axis=0)[None, :]
    blk = jnp.concatenate([s, q, jnp.zeros((6, DIM), jnp.float32)], axis=0)

    @pl.when(pid == 0)
    def _():
        st_ref[...] = blk

    @pl.when(pid > 0)
    def _():
        st_ref[...] += blk


def _mid(h128, a128, kin, w1, b1, w2, b2):
    return pl.pallas_call(
        functools.partial(_mid_body, kin),
        grid=(NBLK,),
        in_specs=[
            pl.BlockSpec((MBLK, LANES), lambda i: (i, 0)),
            pl.BlockSpec((MBLK, LANES), lambda i: (i, 0)),
            pl.BlockSpec((kin, DIM), lambda i: (0, 0)),
            pl.BlockSpec((1, DIM), lambda i: (0, 0)),
            pl.BlockSpec((DIM, DIM), lambda i: (0, 0)),
            pl.BlockSpec((1, DIM), lambda i: (0, 0)),
        ],
        out_specs=[
            pl.BlockSpec((MBLK, DIM), lambda i: (i, 0)),
            pl.BlockSpec((8, DIM), lambda i: (0, 0)),
        ],
        out_shape=[
            jax.ShapeDtypeStruct((NPAD, DIM), jnp.float32),
            jax.ShapeDtypeStruct((8, DIM), jnp.float32),
        ],
    )(h128, a128, w1, b1, w2, b2)


def _stat2_body(r_ref, mu_ref, o_ref):
    pid = pl.program_id(0)
    dlt = r_ref[...] - mu_ref[...]
    rows = lax.broadcasted_iota(jnp.int32, dlt.shape, 0) + pid * MBLK
    dlt = jnp.where(rows < N_NODES, dlt, 0.0)
    blk = jnp.concatenate(
        [jnp.sum(dlt * dlt, axis=0)[None, :],
         jnp.zeros((7, DIM), jnp.float32)], axis=0)

    @pl.when(pid == 0)
    def _():
        o_ref[...] = blk

    @pl.when(pid > 0)
    def _():
        o_ref[...] += blk


def _stat2(r, mu):
    return pl.pallas_call(
        _stat2_body,
        grid=(NBLK,),
        in_specs=[
            pl.BlockSpec((MBLK, DIM), lambda i: (i, 0)),
            pl.BlockSpec((1, DIM), lambda i: (0, 0)),
        ],
        out_specs=pl.BlockSpec((8, DIM), lambda i: (0, 0)),
        out_shape=jax.ShapeDtypeStruct((8, DIM), jnp.float32),
    )(r, mu)


def _bn_body(r_ref, stv_ref, o_ref):
    h = ((r_ref[...] - stv_ref[0:1, :]) / stv_ref[1:2, :]
         ) * stv_ref[2:3, :] + stv_ref[3:4, :]
    o_ref[...] = jnp.concatenate(
        [h, jnp.zeros((h.shape[0], LANES - DIM), jnp.float32)], axis=1)


def _bn_apply(r, stv):
    return pl.pallas_call(
        _bn_body,
        grid=(NBLK,),
        in_specs=[
            pl.BlockSpec((MBLK, DIM), lambda i: (i, 0)),
            pl.BlockSpec((8, DIM), lambda i: (0, 0)),
        ],
        out_specs=pl.BlockSpec((MBLK, LANES), lambda i: (i, 0)),
        out_shape=jax.ShapeDtypeStruct((NPAD, LANES), jnp.float32),
    )(r, stv)


def _pool_body(h_ref, bc_ref, sums_ref):
    pid = pl.program_id(0)
    gids = lax.broadcasted_iota(jnp.int32, (MBLK, N_GRAPHS), 1
                                ).astype(jnp.float32)
    oh = (bc_ref[...] == gids).astype(jnp.float32)
    sums = lax.dot_general(oh, h_ref[:, :DIM], (((0,), (0,)), ((), ())),
                           precision=_HI, preferred_element_type=jnp.float32)

    @pl.when(pid == 0)
    def _():
        sums_ref[...] = sums

    @pl.when(pid > 0)
    def _():
        sums_ref[...] += sums


def _pool(h128, batchcol):
    return pl.pallas_call(
        _pool_body,
        grid=(NBLK,),
        in_specs=[
            pl.BlockSpec((MBLK, LANES), lambda i: (i, 0)),
            pl.BlockSpec((MBLK, 1), lambda i: (i, 0)),
        ],
        out_specs=pl.BlockSpec((N_GRAPHS, DIM), lambda i: (0, 0)),
        out_shape=jax.ShapeDtypeStruct((N_GRAPHS, DIM), jnp.float32),
    )(h128, batchcol)


_BB = 8


def _conv1_body(tgt_ref, wt_ref, o_ref):
    for b in range(_BB):
        t_row = tgt_ref[b, :][None, :]
        vocab = lax.broadcasted_iota(jnp.int32, (VOCAB_T, SEQ_LEN), 0
                                     ).astype(jnp.float32)
        oh = (jnp.broadcast_to(t_row, (VOCAB_T, SEQ_LEN)) == vocab)
        oh = oh.astype(jnp.float32)
        o_ref[b] = jnp.dot(oh, wt_ref[...],
                           preferred_element_type=jnp.float32)


def _conv1(tgtf, wt):
    return pl.pallas_call(
        _conv1_body,
        grid=(N_GRAPHS // _BB,),
        in_specs=[
            pl.BlockSpec((_BB, SEQ_LEN), lambda i: (i, 0)),
            pl.BlockSpec((SEQ_LEN, N_FILTERS * KW), lambda i: (0, 0)),
        ],
        out_specs=pl.BlockSpec((_BB, VOCAB_T, N_FILTERS * KW),
                               lambda i: (i, 0, 0)),
        out_shape=jax.ShapeDtypeStruct((N_GRAPHS, VOCAB_T, N_FILTERS * KW),
                                       jnp.float32),
    )(tgtf, wt)


_MB2 = 1024


def _conv2_body(cr_ref, bm_ref, bias_ref, o_ref):
    o_ref[...] = (jnp.dot(cr_ref[...], bm_ref[...], precision=_HI,
                          preferred_element_type=jnp.float32)
                  + bias_ref[...])


def _conv2(cr, bm, biascol):
    m = cr.shape[0]
    return pl.pallas_call(
        _conv2_body,
        grid=(m // _MB2,),
        in_specs=[
            pl.BlockSpec((_MB2, VOCAB_T * KW), lambda i: (i, 0)),
            pl.BlockSpec((VOCAB_T * KW, CONV_OUT_LEN), lambda i: (0, 0)),
            pl.BlockSpec((_MB2, 1), lambda i: (i, 0)),
        ],
        out_specs=pl.BlockSpec((_MB2, CONV_OUT_LEN), lambda i: (i, 0)),
        out_shape=jax.ShapeDtypeStruct((m, CONV_OUT_LEN), jnp.float32),
    )(cr, bm, biascol)


def _head_body(pooled_ref, xdw_ref, xdb_ref,
               cf_ref, xtw_ref, xtb_ref,
               wa_ref, wb_ref, f1b_ref, f2w_ref, f2b_ref, ow_ref, ob_ref,
               o_ref):
    xd = jnp.maximum(jnp.dot(pooled_ref[...], xdw_ref[...], preferred_element_type=jnp.float32)
                     + xdb_ref[...], 0.0)
    xt = jnp.dot(cf_ref[...], xtw_ref[...], preferred_element_type=jnp.float32) + xtb_ref[...]
    z1 = jnp.maximum(
        jnp.dot(xd, wa_ref[...], preferred_element_type=jnp.float32)
        + jnp.dot(xt, wb_ref[...], preferred_element_type=jnp.float32)
        + f1b_ref[...], 0.0)
    z2 = jnp.maximum(jnp.dot(z1, f2w_ref[...], preferred_element_type=jnp.float32)
                     + f2b_ref[...], 0.0)
    o_ref[...] = jnp.dot(z2, ow_ref[...], preferred_element_type=jnp.float32) + ob_ref[...]


def _head(pooled, xdw, xdb, cf, xtw, xtb,
          wa, wb, f1b, f2w, f2b, ow, ob):
    return pl.pallas_call(
        _head_body,
        out_shape=jax.ShapeDtypeStruct((N_GRAPHS, 1), jnp.float32),
    )(pooled, xdw, xdb, cf, xtw, xtb,
      wa, wb, f1b, f2w, f2b, ow, ob)


def kernel(x, params, edge_index, batch, target):
    f32 = jnp.float32
    h = jnp.pad(x, ((0, NPAD - N_NODES), (0, LANES - D_IN)))
    pad_e = EPAD - N_EDGES
    neg = jnp.full((pad_e,), -1, jnp.int32)
    srcp = jnp.concatenate([edge_index[0], neg])
    dstp = jnp.concatenate([edge_index[1], neg])
    dstq = dstp // S

    def masked(m):
        sm = jnp.where(dstq == m, srcp, -1).reshape(NS, G, CHUNK)
        dm = jnp.where(dstq == m, dstp - m * S, -1).reshape(NS, G, CHUNK)
        return sm, dm

    sq = [masked(m) for m in range(NSLICE)]
    srcm = jnp.stack(
        [jnp.concatenate([sq[2 * k][0], sq[2 * k + 1][0]])
         for k in range(NSLICE // 2)])
    dstm = jnp.stack(
        [jnp.concatenate([sq[2 * k][1], sq[2 * k + 1][1]])
         for k in range(NSLICE // 2)])
    comb = jnp.stack([srcm, dstm]).reshape(2, 2, NC * NS, G, CHUNK)
    comb = jnp.transpose(comb, (2, 3, 0, 1, 4)).reshape(-1, CHUNK)
    batchcol = jnp.pad(batch.astype(f32), (0, NPAD - N_NODES),
                       constant_values=-1.0).reshape(NPAD, 1)

    def row(v):
        return v.reshape(1, -1)

    for l in range(5):
        parts = [_sc_agg(k, comb, h)
                 for k in range(NSLICE // 2)]
        agg = jnp.concatenate(parts, axis=0)
        kin = D_IN if l == 0 else DIM
        r, st = _mid(h, agg, kin, params[f"gin{l}_1_w"],
                     row(params[f"gin{l}_1_b"]),
                     params[f"gin{l}_2_w"], row(params[f"gin{l}_2_b"]))
        del st
        rv = r[:N_NODES]
        mu = jnp.mean(rv, axis=0)
        var = jnp.var(rv, axis=0)
        sq = jnp.sqrt(var + 1e-5)
        stv = jnp.concatenate(
            [row(mu), row(sq), row(params[f"bn{l}_g"]),
             row(params[f"bn{l}_b"]), jnp.zeros((4, DIM), f32)], axis=0)
        h = _bn_apply(r, stv)

    pooled = _pool(h, batchcol)

    wt = jnp.transpose(params["conv_xt_w"], (1, 0, 2)).reshape(
        SEQ_LEN, N_FILTERS * KW)
    c2 = _conv1(target.astype(f32), wt)
    cr = jnp.transpose(c2.reshape(N_GRAPHS, VOCAB_T, N_FILTERS, KW),
                       (0, 2, 1, 3)).reshape(N_GRAPHS * N_FILTERS,
                                             VOCAB_T * KW)
    emb = params["emb_xt"]
    bm = jnp.stack([emb[:, k:k + CONV_OUT_LEN] for k in range(KW)],
                   axis=1).reshape(VOCAB_T * KW, CONV_OUT_LEN)
    biascol = jnp.tile(params["conv_xt_b"], N_GRAPHS).reshape(-1, 1)
    convf = _conv2(cr, bm, biascol).reshape(N_GRAPHS,
                                            N_FILTERS * CONV_OUT_LEN)

    wa = params["fc1_w"][:OUTPUT_DIM]
    wb = params["fc1_w"][OUTPUT_DIM:]
    return _head(pooled,
                 params["fc1_xd_w"], row(params["fc1_xd_b"]),
                 convf, params["fc1_xt_w"], row(params["fc1_xt_b"]),
                 wa, wb, row(params["fc1_b"]),
                 params["fc2_w"], row(params["fc2_b"]),
                 params["out_w"], row(params["out_b"]))
